# Initial kernel scaffold; baseline (speedup 1.0000x reference)
import jax
import jax.numpy as jnp
from jax.experimental import pallas as pl


def kernel(x, edge_index, edge_attr, W1a, b1a, W1b, b1b, W2a, b2a, W2b, b2b, Wlt, blt):
    raise NotImplementedError("write your pallas kernel here")


# speedup vs baseline: 4.8958x; 4.8958x over previous
import functools

import jax
import jax.numpy as jnp
from jax import lax
from jax.experimental import pallas as pl
from jax.experimental.pallas import tpu as pltpu
from jax.experimental.pallas import tpu_sc as plsc

_NC = 2
_NS = 16
_NW = _NC * _NS


def _segsum_sc(table, src, dst, zeros):
  n, d = table.shape
  e = src.shape[0]
  epw = e // _NW
  k = 80
  iters = epw // k
  rps = (n // _NS) // 8 * 8
  rem = n - rps * _NS

  mesh = plsc.VectorSubcoreMesh(
      core_axis_name="c", subcore_axis_name="s",
      num_cores=_NC, num_subcores=_NS)

  @functools.partial(
      pl.kernel,
      mesh=mesh,
      out_type=jax.ShapeDtypeStruct((_NC, n, d), jnp.float32),
      scratch_types=[
          pltpu.VMEM((k,), jnp.int32),
          pltpu.VMEM((k,), jnp.int32),
          pltpu.VMEM((k, d), jnp.float32),
          pltpu.VMEM_SHARED((n, d), jnp.float32),
          pltpu.SemaphoreType.DMA,
      ],
  )
  def seg_kernel(table_hbm, src_hbm, dst_hbm, zeros_hbm, out_hbm,
                 sidx, didx, rows, acc, sem):
    c = lax.axis_index("c")
    s = lax.axis_index("s")
    wid = s * _NC + c

    pltpu.sync_copy(zeros_hbm.at[pl.ds(s * rps, rps)],
                    acc.at[pl.ds(s * rps, rps)])
    if rem:
      @pl.when(s == 0)
      def _():
        pltpu.sync_copy(zeros_hbm.at[pl.ds(rps * _NS, rem)],
                        acc.at[pl.ds(rps * _NS, rem)])
    plsc.subcore_barrier()

    base = wid * epw

    def body(i, carry):
      off = base + i * k
      pltpu.sync_copy(src_hbm.at[pl.ds(off, k)], sidx)
      pltpu.sync_copy(dst_hbm.at[pl.ds(off, k)], didx)
      pltpu.async_copy(table_hbm.at[sidx], rows, sem).wait()
      pltpu.sync_copy(rows, acc.at[didx], add=True)
      return carry

    lax.fori_loop(0, iters, body, 0)
    plsc.subcore_barrier()

    pltpu.sync_copy(acc.at[pl.ds(s * rps, rps)],
                    out_hbm.at[c].at[pl.ds(s * rps, rps)])
    if rem:
      @pl.when(s == 0)
      def _():
        pltpu.sync_copy(acc.at[pl.ds(rps * _NS, rem)],
                        out_hbm.at[c].at[pl.ds(rps * _NS, rem)])

  return seg_kernel(table, src, dst, zeros)


def _layer1_kernel(x_ref, p_ref, wa_ref, ba_ref, wb_ref, bb_ref, o_ref):
  z = x_ref[...] + p_ref[0] + p_ref[1]
  h = jnp.dot(z, wa_ref[...], preferred_element_type=jnp.float32) + ba_ref[...]
  h = jnp.maximum(h, 0.0)
  h = jnp.dot(h, wb_ref[...], preferred_element_type=jnp.float32) + bb_ref[...]
  o_ref[...] = jnp.maximum(h, 0.0)


def _proj_kernel(x_ref, w_ref, o_ref):
  o_ref[...] = jnp.dot(x_ref[...], w_ref[...],
                       preferred_element_type=jnp.float32)


def _final_kernel(u_ref, q_ref, ba_ref, wb_ref, bb_ref, wl_ref, bl_ref,
                  o_ref):
  t = u_ref[...] + q_ref[0] + q_ref[1] + ba_ref[...]
  t = jnp.maximum(t, 0.0)
  h2 = jnp.dot(t, wb_ref[...], preferred_element_type=jnp.float32) + bb_ref[...]
  o_ref[...] = jnp.dot(h2, wl_ref[...],
                       preferred_element_type=jnp.float32) + bl_ref[...]


def kernel(x, edge_index, edge_attr, W1a, b1a, W1b, b1b, W2a, b2a, W2b, b2b,
           Wlt, blt):
  del edge_attr
  n, d_in = x.shape
  hid = W1a.shape[1]
  out_d = Wlt.shape[1]
  src = edge_index[0]
  dst = edge_index[1]
  zeros = jnp.zeros((n, d_in), jnp.float32)

  bn = 2000
  grid = (n // bn,)

  full = lambda shape: pl.BlockSpec(shape, lambda i: tuple(0 for _ in shape))

  parts1 = _segsum_sc(x, src, dst, zeros)

  h1 = pl.pallas_call(
      _layer1_kernel,
      grid=grid,
      in_specs=[pl.BlockSpec((bn, d_in), lambda i: (i, 0)),
                pl.BlockSpec((_NC, bn, d_in), lambda i: (0, i, 0)),
                full((d_in, hid)),
                full((1, hid)),
                full((hid, hid)),
                full((1, hid))],
      out_specs=pl.BlockSpec((bn, hid), lambda i: (i, 0)),
      out_shape=jax.ShapeDtypeStruct((n, hid), jnp.float32),
  )(x, parts1, W1a, b1a.reshape(1, hid), W1b, b1b.reshape(1, hid))

  d2 = W2a.shape[1]
  u = pl.pallas_call(
      _proj_kernel,
      grid=grid,
      in_specs=[pl.BlockSpec((bn, hid), lambda i: (i, 0)),
                full((hid, d2))],
      out_specs=pl.BlockSpec((bn, d2), lambda i: (i, 0)),
      out_shape=jax.ShapeDtypeStruct((n, d2), jnp.float32),
  )(h1, W2a)

  parts2 = _segsum_sc(u, src, dst, zeros)

  out = pl.pallas_call(
      _final_kernel,
      grid=grid,
      in_specs=[pl.BlockSpec((bn, d2), lambda i: (i, 0)),
                pl.BlockSpec((_NC, bn, d2), lambda i: (0, i, 0)),
                full((1, d2)),
                full((W2b.shape[0], W2b.shape[1])),
                full((1, W2b.shape[1])),
                full((Wlt.shape[0], out_d)),
                full((1, out_d))],
      out_specs=pl.BlockSpec((bn, out_d), lambda i: (i, 0)),
      out_shape=jax.ShapeDtypeStruct((n, out_d), jnp.float32),
  )(u, parts2, b2a.reshape(1, -1), W2b, b2b.reshape(1, -1),
    Wlt, blt.reshape(1, -1))

  return out


# speedup vs baseline: 6.2439x; 1.2754x over previous
import functools

import jax
import jax.numpy as jnp
from jax import lax
from jax.experimental import pallas as pl
from jax.experimental.pallas import tpu as pltpu
from jax.experimental.pallas import tpu_sc as plsc

_NC = 2
_NS = 16
_NW = _NC * _NS


def _segsum_sc(table, src, dst, zeros):
  n, d = table.shape
  e = src.shape[0]
  epw = e // _NW
  k = 40
  iters = epw // k
  pairs = iters // 2
  tail = iters - 2 * pairs
  rps = (n // _NS) // 8 * 8
  rem = n - rps * _NS

  src3 = src.reshape(_NW, iters, k)
  dst3 = dst.reshape(_NW, iters, k)

  mesh = plsc.VectorSubcoreMesh(
      core_axis_name="c", subcore_axis_name="s",
      num_cores=_NC, num_subcores=_NS)

  @functools.partial(
      pl.kernel,
      mesh=mesh,
      out_type=jax.ShapeDtypeStruct((_NC, n, d), jnp.float32),
      scratch_types=[
          pltpu.VMEM((2, k), jnp.int32),
          pltpu.VMEM((2, k), jnp.int32),
          pltpu.VMEM((2, k, d), jnp.float32),
          pltpu.VMEM_SHARED((n, d), jnp.float32),
          pltpu.SemaphoreType.DMA,
          pltpu.SemaphoreType.DMA,
          pltpu.SemaphoreType.DMA,
          pltpu.SemaphoreType.DMA,
      ],
  )
  def seg_kernel(table_hbm, src_hbm, dst_hbm, zeros_hbm, out_hbm,
                 sidx, didx, rows, acc, gsem0, gsem1, isem0, isem1):
    c = lax.axis_index("c")
    s = lax.axis_index("s")
    wid = s * _NC + c
    gsems = (gsem0, gsem1)
    isems = (isem0, isem1)

    pltpu.sync_copy(zeros_hbm.at[pl.ds(s * rps, rps)],
                    acc.at[pl.ds(s * rps, rps)])
    if rem:
      @pl.when(s == 0)
      def _():
        pltpu.sync_copy(zeros_hbm.at[pl.ds(rps * _NS, rem)],
                        acc.at[pl.ds(rps * _NS, rem)])

    def idx_fire(ci, b):
      pltpu.async_copy(src_hbm.at[wid, ci], sidx.at[b], isems[b])
      pltpu.async_copy(dst_hbm.at[wid, ci], didx.at[b], isems[b])

    def idx_wait(b):
      pltpu.make_async_copy(src_hbm.at[wid, 0], sidx.at[b], isems[b]).wait()
      pltpu.make_async_copy(dst_hbm.at[wid, 0], didx.at[b], isems[b]).wait()

    def gat_fire(b):
      pltpu.async_copy(table_hbm.at[sidx.at[b]], rows.at[b], gsems[b])

    def gat_wait(b):
      pltpu.make_async_copy(table_hbm.at[sidx.at[0]], rows.at[b],
                            gsems[b]).wait()

    def scat(b):
      pltpu.sync_copy(rows.at[b], acc.at[didx.at[b]], add=True)

    idx_fire(0, 0)
    idx_fire(1, 1)
    plsc.subcore_barrier()
    idx_wait(0)
    gat_fire(0)

    def step(ci, b, nb):
      gat_wait(b)

      @pl.when(ci + 1 < iters)
      def _():
        idx_wait(nb)
        gat_fire(nb)

      scat(b)

      @pl.when(ci + 2 < iters)
      def _():
        idx_fire(ci + 2, b)

    def body(j, carry):
      step(2 * j, 0, 1)
      step(2 * j + 1, 1, 0)
      return carry

    lax.fori_loop(0, pairs, body, 0)
    if tail:
      step(iters - 1, 0, 1)
    plsc.subcore_barrier()

    pltpu.sync_copy(acc.at[pl.ds(s * rps, rps)],
                    out_hbm.at[c].at[pl.ds(s * rps, rps)])
    if rem:
      @pl.when(s == 0)
      def _():
        pltpu.sync_copy(acc.at[pl.ds(rps * _NS, rem)],
                        out_hbm.at[c].at[pl.ds(rps * _NS, rem)])

  return seg_kernel(table, src3, dst3, zeros)


def _layer1_kernel(x_ref, p_ref, wa_ref, ba_ref, wb_ref, bb_ref, o_ref):
  z = x_ref[...] + p_ref[0] + p_ref[1]
  h = jnp.dot(z, wa_ref[...], preferred_element_type=jnp.float32) + ba_ref[...]
  h = jnp.maximum(h, 0.0)
  h = jnp.dot(h, wb_ref[...], preferred_element_type=jnp.float32) + bb_ref[...]
  o_ref[...] = jnp.maximum(h, 0.0)


def _proj_kernel(x_ref, w_ref, o_ref):
  o_ref[...] = jnp.dot(x_ref[...], w_ref[...],
                       preferred_element_type=jnp.float32)


def _final_kernel(u_ref, q_ref, ba_ref, wb_ref, bb_ref, wl_ref, bl_ref,
                  o_ref):
  t = u_ref[...] + q_ref[0] + q_ref[1] + ba_ref[...]
  t = jnp.maximum(t, 0.0)
  h2 = jnp.dot(t, wb_ref[...], preferred_element_type=jnp.float32) + bb_ref[...]
  o_ref[...] = jnp.dot(h2, wl_ref[...],
                       preferred_element_type=jnp.float32) + bl_ref[...]


def kernel(x, edge_index, edge_attr, W1a, b1a, W1b, b1b, W2a, b2a, W2b, b2b,
           Wlt, blt):
  del edge_attr
  n, d_in = x.shape
  hid = W1a.shape[1]
  out_d = Wlt.shape[1]
  src = edge_index[0]
  dst = edge_index[1]
  zeros = jnp.zeros((n, d_in), jnp.float32)

  bn = 2000
  grid = (n // bn,)

  full = lambda shape: pl.BlockSpec(shape, lambda i: tuple(0 for _ in shape))

  parts1 = _segsum_sc(x, src, dst, zeros)

  h1 = pl.pallas_call(
      _layer1_kernel,
      grid=grid,
      in_specs=[pl.BlockSpec((bn, d_in), lambda i: (i, 0)),
                pl.BlockSpec((_NC, bn, d_in), lambda i: (0, i, 0)),
                full((d_in, hid)),
                full((1, hid)),
                full((hid, hid)),
                full((1, hid))],
      out_specs=pl.BlockSpec((bn, hid), lambda i: (i, 0)),
      out_shape=jax.ShapeDtypeStruct((n, hid), jnp.float32),
  )(x, parts1, W1a, b1a.reshape(1, hid), W1b, b1b.reshape(1, hid))

  d2 = W2a.shape[1]
  u = pl.pallas_call(
      _proj_kernel,
      grid=grid,
      in_specs=[pl.BlockSpec((bn, hid), lambda i: (i, 0)),
                full((hid, d2))],
      out_specs=pl.BlockSpec((bn, d2), lambda i: (i, 0)),
      out_shape=jax.ShapeDtypeStruct((n, d2), jnp.float32),
  )(h1, W2a)

  parts2 = _segsum_sc(u, src, dst, zeros)

  out = pl.pallas_call(
      _final_kernel,
      grid=grid,
      in_specs=[pl.BlockSpec((bn, d2), lambda i: (i, 0)),
                pl.BlockSpec((_NC, bn, d2), lambda i: (0, i, 0)),
                full((1, d2)),
                full((W2b.shape[0], W2b.shape[1])),
                full((1, W2b.shape[1])),
                full((Wlt.shape[0], out_d)),
                full((1, out_d))],
      out_specs=pl.BlockSpec((bn, out_d), lambda i: (i, 0)),
      out_shape=jax.ShapeDtypeStruct((n, out_d), jnp.float32),
  )(u, parts2, b2a.reshape(1, -1), W2b, b2b.reshape(1, -1),
    Wlt, blt.reshape(1, -1))

  return out


# speedup vs baseline: 8.7908x; 1.4079x over previous
import functools

import jax
import jax.numpy as jnp
from jax import lax
from jax.experimental import pallas as pl
from jax.experimental.pallas import tpu as pltpu
from jax.experimental.pallas import tpu_sc as plsc

_NC = 2
_NS = 16
_NW = _NC * _NS


def _segsum_sc(table, src, dst, zeros):
  n, d = table.shape
  e = src.shape[0]
  epw = e // _NW
  k = 80
  iters = epw // k
  pairs = iters // 2
  tail = iters - 2 * pairs
  rps = (n // _NS) // 8 * 8
  rem = n - rps * _NS

  src3 = src.reshape(_NW, iters, k)
  dst3 = dst.reshape(_NW, iters, k)

  mesh = plsc.VectorSubcoreMesh(
      core_axis_name="c", subcore_axis_name="s",
      num_cores=_NC, num_subcores=_NS)

  @functools.partial(
      pl.kernel,
      mesh=mesh,
      out_type=jax.ShapeDtypeStruct((_NC, n, d), jnp.float32),
      scratch_types=[
          pltpu.VMEM((2, k), jnp.int32),
          pltpu.VMEM((2, k), jnp.int32),
          pltpu.VMEM((2, k, d), jnp.float32),
          pltpu.VMEM_SHARED((n, d), jnp.float32),
          pltpu.SemaphoreType.DMA,
          pltpu.SemaphoreType.DMA,
          pltpu.SemaphoreType.DMA,
          pltpu.SemaphoreType.DMA,
      ],
  )
  def seg_kernel(table_hbm, src_hbm, dst_hbm, zeros_hbm, out_hbm,
                 sidx, didx, rows, acc, gsem0, gsem1, isem0, isem1):
    c = lax.axis_index("c")
    s = lax.axis_index("s")
    wid = s * _NC + c
    gsems = (gsem0, gsem1)
    isems = (isem0, isem1)

    pltpu.sync_copy(zeros_hbm.at[pl.ds(s * rps, rps)],
                    acc.at[pl.ds(s * rps, rps)])
    if rem:
      @pl.when(s == 0)
      def _():
        pltpu.sync_copy(zeros_hbm.at[pl.ds(rps * _NS, rem)],
                        acc.at[pl.ds(rps * _NS, rem)])

    def idx_fire(ci, b):
      pltpu.async_copy(src_hbm.at[wid, ci], sidx.at[b], isems[b])
      pltpu.async_copy(dst_hbm.at[wid, ci], didx.at[b], isems[b])

    def idx_wait(b):
      pltpu.make_async_copy(src_hbm.at[wid, 0], sidx.at[b], isems[b]).wait()
      pltpu.make_async_copy(dst_hbm.at[wid, 0], didx.at[b], isems[b]).wait()

    def gat_fire(b):
      pltpu.async_copy(table_hbm.at[sidx.at[b]], rows.at[b], gsems[b])

    def gat_wait(b):
      pltpu.make_async_copy(table_hbm.at[sidx.at[0]], rows.at[b],
                            gsems[b]).wait()

    def scat(b):
      pltpu.sync_copy(rows.at[b], acc.at[didx.at[b]], add=True)

    idx_fire(0, 0)
    idx_fire(1, 1)
    plsc.subcore_barrier()
    idx_wait(0)
    gat_fire(0)

    def step(ci, b, nb):
      gat_wait(b)

      @pl.when(ci + 1 < iters)
      def _():
        idx_wait(nb)
        gat_fire(nb)

      scat(b)

      @pl.when(ci + 2 < iters)
      def _():
        idx_fire(ci + 2, b)

    def body(j, carry):
      step(2 * j, 0, 1)
      step(2 * j + 1, 1, 0)
      return carry

    lax.fori_loop(0, pairs, body, 0)
    if tail:
      step(iters - 1, 0, 1)
    plsc.subcore_barrier()

    pltpu.sync_copy(acc.at[pl.ds(s * rps, rps)],
                    out_hbm.at[c].at[pl.ds(s * rps, rps)])
    if rem:
      @pl.when(s == 0)
      def _():
        pltpu.sync_copy(acc.at[pl.ds(rps * _NS, rem)],
                        out_hbm.at[c].at[pl.ds(rps * _NS, rem)])

  return seg_kernel(table, src3, dst3, zeros)


def _layer1_kernel(x_ref, p_ref, wa_ref, ba_ref, wb_ref, bb_ref, o_ref):
  z = x_ref[...] + p_ref[0] + p_ref[1]
  h = jnp.dot(z, wa_ref[...], preferred_element_type=jnp.float32) + ba_ref[...]
  h = jnp.maximum(h, 0.0)
  h = jnp.dot(h, wb_ref[...], preferred_element_type=jnp.float32) + bb_ref[...]
  o_ref[...] = jnp.maximum(h, 0.0)


def _proj_kernel(x_ref, w_ref, o_ref):
  o_ref[...] = jnp.dot(x_ref[...], w_ref[...],
                       preferred_element_type=jnp.float32)


def _final_kernel(u_ref, q_ref, ba_ref, wb_ref, bb_ref, wl_ref, bl_ref,
                  o_ref):
  t = u_ref[...] + q_ref[0] + q_ref[1] + ba_ref[...]
  t = jnp.maximum(t, 0.0)
  h2 = jnp.dot(t, wb_ref[...], preferred_element_type=jnp.float32) + bb_ref[...]
  o_ref[...] = jnp.dot(h2, wl_ref[...],
                       preferred_element_type=jnp.float32) + bl_ref[...]


def kernel(x, edge_index, edge_attr, W1a, b1a, W1b, b1b, W2a, b2a, W2b, b2b,
           Wlt, blt):
  del edge_attr
  n, d_in = x.shape
  hid = W1a.shape[1]
  out_d = Wlt.shape[1]
  src = edge_index[0]
  dst = edge_index[1]
  zeros = jnp.zeros((n, d_in), jnp.float32)

  bn = 2000
  grid = (n // bn,)

  full = lambda shape: pl.BlockSpec(shape, lambda i: tuple(0 for _ in shape))

  parts1 = _segsum_sc(x, src, dst, zeros)

  h1 = pl.pallas_call(
      _layer1_kernel,
      grid=grid,
      in_specs=[pl.BlockSpec((bn, d_in), lambda i: (i, 0)),
                pl.BlockSpec((_NC, bn, d_in), lambda i: (0, i, 0)),
                full((d_in, hid)),
                full((1, hid)),
                full((hid, hid)),
                full((1, hid))],
      out_specs=pl.BlockSpec((bn, hid), lambda i: (i, 0)),
      out_shape=jax.ShapeDtypeStruct((n, hid), jnp.float32),
  )(x, parts1, W1a, b1a.reshape(1, hid), W1b, b1b.reshape(1, hid))

  d2 = W2a.shape[1]
  u = pl.pallas_call(
      _proj_kernel,
      grid=grid,
      in_specs=[pl.BlockSpec((bn, hid), lambda i: (i, 0)),
                full((hid, d2))],
      out_specs=pl.BlockSpec((bn, d2), lambda i: (i, 0)),
      out_shape=jax.ShapeDtypeStruct((n, d2), jnp.float32),
  )(h1, W2a)

  parts2 = _segsum_sc(u, src, dst, zeros)

  out = pl.pallas_call(
      _final_kernel,
      grid=grid,
      in_specs=[pl.BlockSpec((bn, d2), lambda i: (i, 0)),
                pl.BlockSpec((_NC, bn, d2), lambda i: (0, i, 0)),
                full((1, d2)),
                full((W2b.shape[0], W2b.shape[1])),
                full((1, W2b.shape[1])),
                full((Wlt.shape[0], out_d)),
                full((1, out_d))],
      out_specs=pl.BlockSpec((bn, out_d), lambda i: (i, 0)),
      out_shape=jax.ShapeDtypeStruct((n, out_d), jnp.float32),
  )(u, parts2, b2a.reshape(1, -1), W2b, b2b.reshape(1, -1),
    Wlt, blt.reshape(1, -1))

  return out


# speedup vs baseline: 9.5791x; 1.0897x over previous
import functools

import jax
import jax.numpy as jnp
from jax import lax
from jax.experimental import pallas as pl
from jax.experimental.pallas import tpu as pltpu
from jax.experimental.pallas import tpu_sc as plsc

_NC = 2
_NS = 16
_NW = _NC * _NS


def _segsum_sc(table, src, dst, zeros):
  n, d = table.shape
  e = src.shape[0]
  epw = e // _NW
  k = 100
  iters = epw // k
  pairs = iters // 2
  tail = iters - 2 * pairs
  rps = (n // _NS) // 8 * 8
  rem = n - rps * _NS

  src3 = src.reshape(_NW, iters, k)
  dst3 = dst.reshape(_NW, iters, k)

  mesh = plsc.VectorSubcoreMesh(
      core_axis_name="c", subcore_axis_name="s",
      num_cores=_NC, num_subcores=_NS)

  @functools.partial(
      pl.kernel,
      mesh=mesh,
      out_type=jax.ShapeDtypeStruct((_NC, n, d), jnp.float32),
      scratch_types=[
          pltpu.VMEM((2, k), jnp.int32),
          pltpu.VMEM((2, k), jnp.int32),
          pltpu.VMEM((2, k, d), jnp.float32),
          pltpu.VMEM_SHARED((n, d), jnp.float32),
          pltpu.SemaphoreType.DMA,
          pltpu.SemaphoreType.DMA,
          pltpu.SemaphoreType.DMA,
          pltpu.SemaphoreType.DMA,
      ],
  )
  def seg_kernel(table_hbm, src_hbm, dst_hbm, zeros_hbm, out_hbm,
                 sidx, didx, rows, acc, gsem0, gsem1, isem0, isem1):
    c = lax.axis_index("c")
    s = lax.axis_index("s")
    wid = s * _NC + c
    gsems = (gsem0, gsem1)
    isems = (isem0, isem1)

    pltpu.sync_copy(zeros_hbm.at[pl.ds(s * rps, rps)],
                    acc.at[pl.ds(s * rps, rps)])
    if rem:
      @pl.when(s == 0)
      def _():
        pltpu.sync_copy(zeros_hbm.at[pl.ds(rps * _NS, rem)],
                        acc.at[pl.ds(rps * _NS, rem)])

    def idx_fire(ci, b):
      pltpu.async_copy(src_hbm.at[wid, ci], sidx.at[b], isems[b])
      pltpu.async_copy(dst_hbm.at[wid, ci], didx.at[b], isems[b])

    def idx_wait(b):
      pltpu.make_async_copy(src_hbm.at[wid, 0], sidx.at[b], isems[b]).wait()
      pltpu.make_async_copy(dst_hbm.at[wid, 0], didx.at[b], isems[b]).wait()

    def gat_fire(b):
      pltpu.async_copy(table_hbm.at[sidx.at[b]], rows.at[b], gsems[b])

    def gat_wait(b):
      pltpu.make_async_copy(table_hbm.at[sidx.at[0]], rows.at[b],
                            gsems[b]).wait()

    def scat(b):
      pltpu.sync_copy(rows.at[b], acc.at[didx.at[b]], add=True)

    idx_fire(0, 0)
    idx_fire(1, 1)
    plsc.subcore_barrier()
    idx_wait(0)
    gat_fire(0)

    def step(ci, b, nb):
      gat_wait(b)

      @pl.when(ci + 1 < iters)
      def _():
        idx_wait(nb)
        gat_fire(nb)

      scat(b)

      @pl.when(ci + 2 < iters)
      def _():
        idx_fire(ci + 2, b)

    def body(j, carry):
      step(2 * j, 0, 1)
      step(2 * j + 1, 1, 0)
      return carry

    lax.fori_loop(0, pairs, body, 0)
    if tail:
      step(iters - 1, 0, 1)
    plsc.subcore_barrier()

    pltpu.sync_copy(acc.at[pl.ds(s * rps, rps)],
                    out_hbm.at[c].at[pl.ds(s * rps, rps)])
    if rem:
      @pl.when(s == 0)
      def _():
        pltpu.sync_copy(acc.at[pl.ds(rps * _NS, rem)],
                        out_hbm.at[c].at[pl.ds(rps * _NS, rem)])

  return seg_kernel(table, src3, dst3, zeros)


def _layer1_kernel(x_ref, p_ref, wa_ref, ba_ref, wb_ref, bb_ref, o_ref):
  z = x_ref[...] + p_ref[0] + p_ref[1]
  h = jnp.dot(z, wa_ref[...], preferred_element_type=jnp.float32) + ba_ref[...]
  h = jnp.maximum(h, 0.0)
  h = jnp.dot(h, wb_ref[...], preferred_element_type=jnp.float32) + bb_ref[...]
  o_ref[...] = jnp.maximum(h, 0.0)


def _proj_kernel(x_ref, w_ref, o_ref):
  o_ref[...] = jnp.dot(x_ref[...], w_ref[...],
                       preferred_element_type=jnp.float32)


def _final_kernel(u_ref, q_ref, ba_ref, wb_ref, bb_ref, wl_ref, bl_ref,
                  o_ref):
  t = u_ref[...] + q_ref[0] + q_ref[1] + ba_ref[...]
  t = jnp.maximum(t, 0.0)
  h2 = jnp.dot(t, wb_ref[...], preferred_element_type=jnp.float32) + bb_ref[...]
  o_ref[...] = jnp.dot(h2, wl_ref[...],
                       preferred_element_type=jnp.float32) + bl_ref[...]


def kernel(x, edge_index, edge_attr, W1a, b1a, W1b, b1b, W2a, b2a, W2b, b2b,
           Wlt, blt):
  del edge_attr
  n, d_in = x.shape
  hid = W1a.shape[1]
  out_d = Wlt.shape[1]
  src = edge_index[0]
  dst = edge_index[1]
  zeros = jnp.zeros((n, d_in), jnp.float32)

  bn = 2000
  grid = (n // bn,)

  full = lambda shape: pl.BlockSpec(shape, lambda i: tuple(0 for _ in shape))

  parts1 = _segsum_sc(x, src, dst, zeros)

  h1 = pl.pallas_call(
      _layer1_kernel,
      grid=grid,
      in_specs=[pl.BlockSpec((bn, d_in), lambda i: (i, 0)),
                pl.BlockSpec((_NC, bn, d_in), lambda i: (0, i, 0)),
                full((d_in, hid)),
                full((1, hid)),
                full((hid, hid)),
                full((1, hid))],
      out_specs=pl.BlockSpec((bn, hid), lambda i: (i, 0)),
      out_shape=jax.ShapeDtypeStruct((n, hid), jnp.float32),
  )(x, parts1, W1a, b1a.reshape(1, hid), W1b, b1b.reshape(1, hid))

  d2 = W2a.shape[1]
  u = pl.pallas_call(
      _proj_kernel,
      grid=grid,
      in_specs=[pl.BlockSpec((bn, hid), lambda i: (i, 0)),
                full((hid, d2))],
      out_specs=pl.BlockSpec((bn, d2), lambda i: (i, 0)),
      out_shape=jax.ShapeDtypeStruct((n, d2), jnp.float32),
  )(h1, W2a)

  parts2 = _segsum_sc(u, src, dst, zeros)

  out = pl.pallas_call(
      _final_kernel,
      grid=grid,
      in_specs=[pl.BlockSpec((bn, d2), lambda i: (i, 0)),
                pl.BlockSpec((_NC, bn, d2), lambda i: (0, i, 0)),
                full((1, d2)),
                full((W2b.shape[0], W2b.shape[1])),
                full((1, W2b.shape[1])),
                full((Wlt.shape[0], out_d)),
                full((1, out_d))],
      out_specs=pl.BlockSpec((bn, out_d), lambda i: (i, 0)),
      out_shape=jax.ShapeDtypeStruct((n, out_d), jnp.float32),
  )(u, parts2, b2a.reshape(1, -1), W2b, b2b.reshape(1, -1),
    Wlt, blt.reshape(1, -1))

  return out


# speedup vs baseline: 10.2083x; 1.0657x over previous
import functools

import jax
import jax.numpy as jnp
from jax import lax
from jax.experimental import pallas as pl
from jax.experimental.pallas import tpu as pltpu
from jax.experimental.pallas import tpu_sc as plsc

_NC = 2
_NS = 16
_NW = _NC * _NS


def _segsum_sc(table, src, dst, zeros):
  n, d = table.shape
  e = src.shape[0]
  epw = e // _NW
  k = 125
  iters = epw // k
  pairs = iters // 2
  tail = iters - 2 * pairs
  rps = (n // _NS) // 8 * 8
  rem = n - rps * _NS

  src3 = src.reshape(_NW, iters, k)
  dst3 = dst.reshape(_NW, iters, k)

  mesh = plsc.VectorSubcoreMesh(
      core_axis_name="c", subcore_axis_name="s",
      num_cores=_NC, num_subcores=_NS)

  @functools.partial(
      pl.kernel,
      mesh=mesh,
      out_type=jax.ShapeDtypeStruct((_NC, n, d), jnp.float32),
      scratch_types=[
          pltpu.VMEM((2, k), jnp.int32),
          pltpu.VMEM((2, k), jnp.int32),
          pltpu.VMEM((2, k, d), jnp.float32),
          pltpu.VMEM_SHARED((n, d), jnp.float32),
          pltpu.SemaphoreType.DMA,
          pltpu.SemaphoreType.DMA,
          pltpu.SemaphoreType.DMA,
          pltpu.SemaphoreType.DMA,
      ],
  )
  def seg_kernel(table_hbm, src_hbm, dst_hbm, zeros_hbm, out_hbm,
                 sidx, didx, rows, acc, gsem0, gsem1, isem0, isem1):
    c = lax.axis_index("c")
    s = lax.axis_index("s")
    wid = s * _NC + c
    gsems = (gsem0, gsem1)
    isems = (isem0, isem1)

    pltpu.sync_copy(zeros_hbm.at[pl.ds(s * rps, rps)],
                    acc.at[pl.ds(s * rps, rps)])
    if rem:
      @pl.when(s == 0)
      def _():
        pltpu.sync_copy(zeros_hbm.at[pl.ds(rps * _NS, rem)],
                        acc.at[pl.ds(rps * _NS, rem)])

    def idx_fire(ci, b):
      pltpu.async_copy(src_hbm.at[wid, ci], sidx.at[b], isems[b])
      pltpu.async_copy(dst_hbm.at[wid, ci], didx.at[b], isems[b])

    def idx_wait(b):
      pltpu.make_async_copy(src_hbm.at[wid, 0], sidx.at[b], isems[b]).wait()
      pltpu.make_async_copy(dst_hbm.at[wid, 0], didx.at[b], isems[b]).wait()

    def gat_fire(b):
      pltpu.async_copy(table_hbm.at[sidx.at[b]], rows.at[b], gsems[b])

    def gat_wait(b):
      pltpu.make_async_copy(table_hbm.at[sidx.at[0]], rows.at[b],
                            gsems[b]).wait()

    def scat(b):
      pltpu.sync_copy(rows.at[b], acc.at[didx.at[b]], add=True)

    idx_fire(0, 0)
    idx_fire(1, 1)
    plsc.subcore_barrier()
    idx_wait(0)
    gat_fire(0)

    def step(ci, b, nb):
      gat_wait(b)

      @pl.when(ci + 1 < iters)
      def _():
        idx_wait(nb)
        gat_fire(nb)

      scat(b)

      @pl.when(ci + 2 < iters)
      def _():
        idx_fire(ci + 2, b)

    def body(j, carry):
      step(2 * j, 0, 1)
      step(2 * j + 1, 1, 0)
      return carry

    lax.fori_loop(0, pairs, body, 0)
    if tail:
      step(iters - 1, 0, 1)
    plsc.subcore_barrier()

    pltpu.sync_copy(acc.at[pl.ds(s * rps, rps)],
                    out_hbm.at[c].at[pl.ds(s * rps, rps)])
    if rem:
      @pl.when(s == 0)
      def _():
        pltpu.sync_copy(acc.at[pl.ds(rps * _NS, rem)],
                        out_hbm.at[c].at[pl.ds(rps * _NS, rem)])

  return seg_kernel(table, src3, dst3, zeros)


def _layer1_kernel(x_ref, p_ref, wa_ref, ba_ref, wb_ref, bb_ref, o_ref):
  z = x_ref[...] + p_ref[0] + p_ref[1]
  h = jnp.dot(z, wa_ref[...], preferred_element_type=jnp.float32) + ba_ref[...]
  h = jnp.maximum(h, 0.0)
  h = jnp.dot(h, wb_ref[...], preferred_element_type=jnp.float32) + bb_ref[...]
  o_ref[...] = jnp.maximum(h, 0.0)


def _proj_kernel(x_ref, w_ref, o_ref):
  o_ref[...] = jnp.dot(x_ref[...], w_ref[...],
                       preferred_element_type=jnp.float32)


def _final_kernel(u_ref, q_ref, ba_ref, wb_ref, bb_ref, wl_ref, bl_ref,
                  o_ref):
  t = u_ref[...] + q_ref[0] + q_ref[1] + ba_ref[...]
  t = jnp.maximum(t, 0.0)
  h2 = jnp.dot(t, wb_ref[...], preferred_element_type=jnp.float32) + bb_ref[...]
  o_ref[...] = jnp.dot(h2, wl_ref[...],
                       preferred_element_type=jnp.float32) + bl_ref[...]


def kernel(x, edge_index, edge_attr, W1a, b1a, W1b, b1b, W2a, b2a, W2b, b2b,
           Wlt, blt):
  del edge_attr
  n, d_in = x.shape
  hid = W1a.shape[1]
  out_d = Wlt.shape[1]
  src = edge_index[0]
  dst = edge_index[1]
  zeros = jnp.zeros((n, d_in), jnp.float32)

  bn = 2000
  grid = (n // bn,)

  full = lambda shape: pl.BlockSpec(shape, lambda i: tuple(0 for _ in shape))

  parts1 = _segsum_sc(x, src, dst, zeros)

  h1 = pl.pallas_call(
      _layer1_kernel,
      grid=grid,
      in_specs=[pl.BlockSpec((bn, d_in), lambda i: (i, 0)),
                pl.BlockSpec((_NC, bn, d_in), lambda i: (0, i, 0)),
                full((d_in, hid)),
                full((1, hid)),
                full((hid, hid)),
                full((1, hid))],
      out_specs=pl.BlockSpec((bn, hid), lambda i: (i, 0)),
      out_shape=jax.ShapeDtypeStruct((n, hid), jnp.float32),
  )(x, parts1, W1a, b1a.reshape(1, hid), W1b, b1b.reshape(1, hid))

  d2 = W2a.shape[1]
  u = pl.pallas_call(
      _proj_kernel,
      grid=grid,
      in_specs=[pl.BlockSpec((bn, hid), lambda i: (i, 0)),
                full((hid, d2))],
      out_specs=pl.BlockSpec((bn, d2), lambda i: (i, 0)),
      out_shape=jax.ShapeDtypeStruct((n, d2), jnp.float32),
  )(h1, W2a)

  parts2 = _segsum_sc(u, src, dst, zeros)

  out = pl.pallas_call(
      _final_kernel,
      grid=grid,
      in_specs=[pl.BlockSpec((bn, d2), lambda i: (i, 0)),
                pl.BlockSpec((_NC, bn, d2), lambda i: (0, i, 0)),
                full((1, d2)),
                full((W2b.shape[0], W2b.shape[1])),
                full((1, W2b.shape[1])),
                full((Wlt.shape[0], out_d)),
                full((1, out_d))],
      out_specs=pl.BlockSpec((bn, out_d), lambda i: (i, 0)),
      out_shape=jax.ShapeDtypeStruct((n, out_d), jnp.float32),
  )(u, parts2, b2a.reshape(1, -1), W2b, b2b.reshape(1, -1),
    Wlt, blt.reshape(1, -1))

  return out


# speedup vs baseline: 10.4359x; 1.0223x over previous
import functools

import jax
import jax.numpy as jnp
from jax import lax
from jax.experimental import pallas as pl
from jax.experimental.pallas import tpu as pltpu
from jax.experimental.pallas import tpu_sc as plsc

_NC = 2
_NS = 16
_NW = _NC * _NS


def _segsum_sc(table, src, dst, zeros):
  n, d = table.shape
  e = src.shape[0]
  epw = e // _NW
  k = 100
  nb = 3
  iters = epw // k
  triples = iters // nb
  tail = iters - nb * triples
  rps = (n // _NS) // 8 * 8
  rem = n - rps * _NS

  src3 = src.reshape(_NW, iters, k)
  dst3 = dst.reshape(_NW, iters, k)

  mesh = plsc.VectorSubcoreMesh(
      core_axis_name="c", subcore_axis_name="s",
      num_cores=_NC, num_subcores=_NS)

  @functools.partial(
      pl.kernel,
      mesh=mesh,
      out_type=jax.ShapeDtypeStruct((_NC, n, d), jnp.float32),
      scratch_types=[
          pltpu.VMEM((3, k), jnp.int32),
          pltpu.VMEM((3, k), jnp.int32),
          pltpu.VMEM((3, k, d), jnp.float32),
          pltpu.VMEM_SHARED((n, d), jnp.float32),
          pltpu.SemaphoreType.DMA,
          pltpu.SemaphoreType.DMA,
          pltpu.SemaphoreType.DMA,
          pltpu.SemaphoreType.DMA,
          pltpu.SemaphoreType.DMA,
          pltpu.SemaphoreType.DMA,
      ],
  )
  def seg_kernel(table_hbm, src_hbm, dst_hbm, zeros_hbm, out_hbm,
                 sidx, didx, rows, acc, gsem0, gsem1, gsem2,
                 isem0, isem1, isem2):
    c = lax.axis_index("c")
    s = lax.axis_index("s")
    wid = s * _NC + c
    gsems = (gsem0, gsem1, gsem2)
    isems = (isem0, isem1, isem2)

    pltpu.sync_copy(zeros_hbm.at[pl.ds(s * rps, rps)],
                    acc.at[pl.ds(s * rps, rps)])
    if rem:
      @pl.when(s == 0)
      def _():
        pltpu.sync_copy(zeros_hbm.at[pl.ds(rps * _NS, rem)],
                        acc.at[pl.ds(rps * _NS, rem)])

    def idx_fire(ci, b):
      pltpu.async_copy(src_hbm.at[wid, ci], sidx.at[b], isems[b])
      pltpu.async_copy(dst_hbm.at[wid, ci], didx.at[b], isems[b])

    def idx_wait(b):
      pltpu.make_async_copy(src_hbm.at[wid, 0], sidx.at[b], isems[b]).wait()
      pltpu.make_async_copy(dst_hbm.at[wid, 0], didx.at[b], isems[b]).wait()

    def gat_fire(b):
      pltpu.async_copy(table_hbm.at[sidx.at[b]], rows.at[b], gsems[b])

    def gat_wait(b):
      pltpu.make_async_copy(table_hbm.at[sidx.at[0]], rows.at[b],
                            gsems[b]).wait()

    def scat(b):
      pltpu.sync_copy(rows.at[b], acc.at[didx.at[b]], add=True)

    idx_fire(0, 0)
    idx_fire(1, 1)
    idx_fire(2, 2)
    plsc.subcore_barrier()
    idx_wait(0)
    gat_fire(0)
    idx_wait(1)
    gat_fire(1)

    def step(ci, b):
      ahead = (b + 2) % 3

      @pl.when(ci + 2 < iters)
      def _():
        idx_wait(ahead)
        gat_fire(ahead)

      gat_wait(b)
      scat(b)

      @pl.when(ci + 3 < iters)
      def _():
        idx_fire(ci + 3, b)

    def body(j, carry):
      step(3 * j, 0)
      step(3 * j + 1, 1)
      step(3 * j + 2, 2)
      return carry

    lax.fori_loop(0, triples, body, 0)
    for t in range(tail):
      step(nb * triples + t, t)
    plsc.subcore_barrier()

    pltpu.sync_copy(acc.at[pl.ds(s * rps, rps)],
                    out_hbm.at[c].at[pl.ds(s * rps, rps)])
    if rem:
      @pl.when(s == 0)
      def _():
        pltpu.sync_copy(acc.at[pl.ds(rps * _NS, rem)],
                        out_hbm.at[c].at[pl.ds(rps * _NS, rem)])

  return seg_kernel(table, src3, dst3, zeros)


def _layer1_kernel(x_ref, p_ref, wa_ref, ba_ref, wb_ref, bb_ref, o_ref):
  z = x_ref[...] + p_ref[0] + p_ref[1]
  h = jnp.dot(z, wa_ref[...], preferred_element_type=jnp.float32) + ba_ref[...]
  h = jnp.maximum(h, 0.0)
  h = jnp.dot(h, wb_ref[...], preferred_element_type=jnp.float32) + bb_ref[...]
  o_ref[...] = jnp.maximum(h, 0.0)


def _proj_kernel(x_ref, w_ref, o_ref):
  o_ref[...] = jnp.dot(x_ref[...], w_ref[...],
                       preferred_element_type=jnp.float32)


def _final_kernel(u_ref, q_ref, ba_ref, wb_ref, bb_ref, wl_ref, bl_ref,
                  o_ref):
  t = u_ref[...] + q_ref[0] + q_ref[1] + ba_ref[...]
  t = jnp.maximum(t, 0.0)
  h2 = jnp.dot(t, wb_ref[...], preferred_element_type=jnp.float32) + bb_ref[...]
  o_ref[...] = jnp.dot(h2, wl_ref[...],
                       preferred_element_type=jnp.float32) + bl_ref[...]


def kernel(x, edge_index, edge_attr, W1a, b1a, W1b, b1b, W2a, b2a, W2b, b2b,
           Wlt, blt):
  del edge_attr
  n, d_in = x.shape
  hid = W1a.shape[1]
  out_d = Wlt.shape[1]
  src = edge_index[0]
  dst = edge_index[1]
  zeros = jnp.zeros((n, d_in), jnp.float32)

  bn = 2000
  grid = (n // bn,)

  full = lambda shape: pl.BlockSpec(shape, lambda i: tuple(0 for _ in shape))

  parts1 = _segsum_sc(x, src, dst, zeros)

  h1 = pl.pallas_call(
      _layer1_kernel,
      grid=grid,
      in_specs=[pl.BlockSpec((bn, d_in), lambda i: (i, 0)),
                pl.BlockSpec((_NC, bn, d_in), lambda i: (0, i, 0)),
                full((d_in, hid)),
                full((1, hid)),
                full((hid, hid)),
                full((1, hid))],
      out_specs=pl.BlockSpec((bn, hid), lambda i: (i, 0)),
      out_shape=jax.ShapeDtypeStruct((n, hid), jnp.float32),
  )(x, parts1, W1a, b1a.reshape(1, hid), W1b, b1b.reshape(1, hid))

  d2 = W2a.shape[1]
  u = pl.pallas_call(
      _proj_kernel,
      grid=grid,
      in_specs=[pl.BlockSpec((bn, hid), lambda i: (i, 0)),
                full((hid, d2))],
      out_specs=pl.BlockSpec((bn, d2), lambda i: (i, 0)),
      out_shape=jax.ShapeDtypeStruct((n, d2), jnp.float32),
  )(h1, W2a)

  parts2 = _segsum_sc(u, src, dst, zeros)

  out = pl.pallas_call(
      _final_kernel,
      grid=grid,
      in_specs=[pl.BlockSpec((bn, d2), lambda i: (i, 0)),
                pl.BlockSpec((_NC, bn, d2), lambda i: (0, i, 0)),
                full((1, d2)),
                full((W2b.shape[0], W2b.shape[1])),
                full((1, W2b.shape[1])),
                full((Wlt.shape[0], out_d)),
                full((1, out_d))],
      out_specs=pl.BlockSpec((bn, out_d), lambda i: (i, 0)),
      out_shape=jax.ShapeDtypeStruct((n, out_d), jnp.float32),
  )(u, parts2, b2a.reshape(1, -1), W2b, b2b.reshape(1, -1),
    Wlt, blt.reshape(1, -1))

  return out
